# baseline (device time: 62072 ns/iter reference)
import jax
import jax.numpy as jnp
from jax import lax
from jax.experimental import pallas as pl
from jax.experimental.pallas import tpu as pltpu

N_DEV = 32
N_LAYERS = 3
N_HALF = 2
N_COL = 2


def kernel(x, Win0, Wout0, Win1, Wout1, Win2, Wout2):
    b, d = x.shape
    hb = b // N_HALF
    sub = hb // N_DEV
    dc = d // N_COL

    def body(x_ref, win0_ref, wout0_ref, win1_ref, wout1_ref, win2_ref,
             wout2_ref, out_ref, pbuf_ref, gbuf_ref, stage_ref,
             rs_send, rs_recv, ag_send, ag_recv):
        me = lax.axis_index("i")

        barrier = pltpu.get_barrier_semaphore()
        for t in range(1, N_DEV):
            pl.semaphore_signal(
                barrier, inc=1,
                device_id=((me + t) % N_DEV,),
                device_id_type=pl.DeviceIdType.MESH,
            )
        pl.semaphore_wait(barrier, N_DEV - 1)

        def recv_wait(dst_ref, recv_sem):
            pltpu.make_async_remote_copy(
                src_ref=dst_ref, dst_ref=dst_ref,
                send_sem=recv_sem, recv_sem=recv_sem,
                device_id=(me,), device_id_type=pl.DeviceIdType.MESH,
            ).wait_recv()

        weights = [
            (win0_ref, wout0_ref),
            (win1_ref, wout1_ref),
            (win2_ref, wout2_ref),
        ]
        prev_rs = [[[], []], [[], []]]
        prev_ag = [[[], []], [[], []]]

        for layer, (win_ref, wout_ref) in enumerate(weights):
            cur_rs = [[[], []], [[], []]]
            for half in range(N_HALF):
                row0 = half * hb
                my_off = row0 + me * sub
                if layer > 0:
                    for col in range(N_COL):
                        for t in range(1, N_DEV):
                            recv_wait(
                                gbuf_ref.at[col, pl.ds(my_off, sub), :],
                                ag_recv.at[half, col, t - 1],
                            )
                    xh = jnp.concatenate(
                        [gbuf_ref[0, pl.ds(row0, hb), :],
                         gbuf_ref[1, pl.ds(row0, hb), :]], axis=1)
                else:
                    xh = x_ref[pl.ds(row0, hb), :]
                for col in range(N_COL):
                    for rdma in prev_rs[half][col]:
                        rdma.wait_send()

                h = jnp.maximum(
                    jnp.dot(xh, win_ref[...],
                            preferred_element_type=jnp.float32),
                    0.0,
                )
                p = jnp.dot(h, wout_ref[...],
                            preferred_element_type=jnp.float32)
                pbuf_ref[0, pl.ds(row0, hb), :] = p[:, :dc]
                pbuf_ref[1, pl.ds(row0, hb), :] = p[:, dc:]

                for col in range(N_COL):
                    for t in range(1, N_DEV):
                        dst = (me + t) % N_DEV
                        rdma = pltpu.make_async_remote_copy(
                            src_ref=pbuf_ref.at[col,
                                                pl.ds(row0 + dst * sub, sub),
                                                :],
                            dst_ref=stage_ref.at[half, col, t - 1],
                            send_sem=rs_send.at[half, col, t - 1],
                            recv_sem=rs_recv.at[half, col, t - 1],
                            device_id=(dst,),
                            device_id_type=pl.DeviceIdType.MESH,
                        )
                        rdma.start()
                        cur_rs[half][col].append(rdma)

            cur_ag = [[[], []], [[], []]]
            for half in range(N_HALF):
                row0 = half * hb
                my_off = row0 + me * sub
                for col in range(N_COL):
                    for t in range(1, N_DEV):
                        recv_wait(stage_ref.at[half, col, t - 1],
                                  rs_recv.at[half, col, t - 1])
                    for rdma in prev_ag[half][col]:
                        rdma.wait_send()
                    gbuf_ref[col, pl.ds(my_off, sub), :] = (
                        pbuf_ref[col, pl.ds(my_off, sub), :]
                        + jnp.sum(stage_ref[half, col], axis=0)
                    )
                    for t in range(1, N_DEV):
                        dst = (me + t) % N_DEV
                        rdma = pltpu.make_async_remote_copy(
                            src_ref=gbuf_ref.at[col, pl.ds(my_off, sub), :],
                            dst_ref=gbuf_ref.at[col, pl.ds(my_off, sub), :],
                            send_sem=ag_send.at[half, col, t - 1],
                            recv_sem=ag_recv.at[half, col, t - 1],
                            device_id=(dst,),
                            device_id_type=pl.DeviceIdType.MESH,
                        )
                        rdma.start()
                        cur_ag[half][col].append(rdma)

            prev_rs = cur_rs
            prev_ag = cur_ag

        for half in range(N_HALF):
            my_off = half * hb + me * sub
            for col in range(N_COL):
                for t in range(1, N_DEV):
                    recv_wait(gbuf_ref.at[col, pl.ds(my_off, sub), :],
                              ag_recv.at[half, col, t - 1])
                for rdma in prev_rs[half][col]:
                    rdma.wait_send()
                for rdma in prev_ag[half][col]:
                    rdma.wait_send()
        out_ref[:, :dc] = gbuf_ref[0]
        out_ref[:, dc:] = gbuf_ref[1]

    return pl.pallas_call(
        body,
        out_shape=jax.ShapeDtypeStruct((b, d), jnp.float32),
        in_specs=[pl.BlockSpec(memory_space=pltpu.VMEM)] * 7,
        out_specs=pl.BlockSpec(memory_space=pltpu.VMEM),
        scratch_shapes=[
            pltpu.VMEM((N_COL, b, dc), jnp.float32),
            pltpu.VMEM((N_COL, b, dc), jnp.float32),
            pltpu.VMEM((N_HALF, N_COL, N_DEV - 1, sub, dc), jnp.float32),
            pltpu.SemaphoreType.DMA((N_HALF, N_COL, N_DEV - 1)),
            pltpu.SemaphoreType.DMA((N_HALF, N_COL, N_DEV - 1)),
            pltpu.SemaphoreType.DMA((N_HALF, N_COL, N_DEV - 1)),
            pltpu.SemaphoreType.DMA((N_HALF, N_COL, N_DEV - 1)),
        ],
        compiler_params=pltpu.CompilerParams(collective_id=0),
    )(x, Win0, Wout0, Win1, Wout1, Win2, Wout2)


# device time: 59131 ns/iter; 1.0497x vs baseline; 1.0497x over previous
import jax
import jax.numpy as jnp
from jax import lax
from jax.experimental import pallas as pl
from jax.experimental.pallas import tpu as pltpu

N_DEV = 32
N_LAYERS = 3
N_HALF = 2


def kernel(x, Win0, Wout0, Win1, Wout1, Win2, Wout2):
    b, d = x.shape
    hb = b // N_HALF
    sub = hb // N_DEV

    def body(x_ref, win0_ref, wout0_ref, win1_ref, wout1_ref, win2_ref,
             wout2_ref, out_ref, stage_ref,
             rs_send, rs_recv, ag_send, ag_recv):
        me = lax.axis_index("i")

        barrier = pltpu.get_barrier_semaphore()
        for t in range(1, N_DEV):
            pl.semaphore_signal(
                barrier, inc=1,
                device_id=((me + t) % N_DEV,),
                device_id_type=pl.DeviceIdType.MESH,
            )
        pl.semaphore_wait(barrier, N_DEV - 1)

        def recv_wait(dst_ref, recv_sem):
            pltpu.make_async_remote_copy(
                src_ref=dst_ref, dst_ref=dst_ref,
                send_sem=recv_sem, recv_sem=recv_sem,
                device_id=(me,), device_id_type=pl.DeviceIdType.MESH,
            ).wait_recv()

        weights = [
            (win0_ref, wout0_ref),
            (win1_ref, wout1_ref),
            (win2_ref, wout2_ref),
        ]
        prev_rs = [[], []]
        prev_ag = [[], []]

        for layer, (win_ref, wout_ref) in enumerate(weights):
            cur_rs = [[], []]
            for half in range(N_HALF):
                row0 = half * hb
                my_off = row0 + me * sub
                if layer > 0:
                    for t in range(1, N_DEV):
                        recv_wait(
                            out_ref.at[pl.ds(my_off, sub), :],
                            ag_recv.at[half, t - 1],
                        )
                    xh = out_ref[pl.ds(row0, hb), :]
                else:
                    xh = x_ref[pl.ds(row0, hb), :]
                for rdma in prev_rs[half]:
                    rdma.wait_send()
                for rdma in prev_ag[half]:
                    rdma.wait_send()

                h = jnp.maximum(
                    jnp.dot(xh, win_ref[...],
                            preferred_element_type=jnp.float32),
                    0.0,
                )
                out_ref[pl.ds(row0, hb), :] = jnp.dot(
                    h, wout_ref[...], preferred_element_type=jnp.float32
                )

                for t in range(1, N_DEV):
                    dst = (me + t) % N_DEV
                    rdma = pltpu.make_async_remote_copy(
                        src_ref=out_ref.at[pl.ds(row0 + dst * sub, sub), :],
                        dst_ref=stage_ref.at[half, t - 1],
                        send_sem=rs_send.at[half, t - 1],
                        recv_sem=rs_recv.at[half, t - 1],
                        device_id=(dst,),
                        device_id_type=pl.DeviceIdType.MESH,
                    )
                    rdma.start()
                    cur_rs[half].append(rdma)

            cur_ag = [[], []]
            for half in range(N_HALF):
                row0 = half * hb
                my_off = row0 + me * sub
                for t in range(1, N_DEV):
                    recv_wait(stage_ref.at[half, t - 1],
                              rs_recv.at[half, t - 1])
                out_ref[pl.ds(my_off, sub), :] = (
                    out_ref[pl.ds(my_off, sub), :]
                    + jnp.sum(stage_ref[half], axis=0)
                )
                for t in range(1, N_DEV):
                    dst = (me + t) % N_DEV
                    rdma = pltpu.make_async_remote_copy(
                        src_ref=out_ref.at[pl.ds(my_off, sub), :],
                        dst_ref=out_ref.at[pl.ds(my_off, sub), :],
                        send_sem=ag_send.at[half, t - 1],
                        recv_sem=ag_recv.at[half, t - 1],
                        device_id=(dst,),
                        device_id_type=pl.DeviceIdType.MESH,
                    )
                    rdma.start()
                    cur_ag[half].append(rdma)

            prev_rs = cur_rs
            prev_ag = cur_ag

        for half in range(N_HALF):
            my_off = half * hb + me * sub
            for t in range(1, N_DEV):
                recv_wait(out_ref.at[pl.ds(my_off, sub), :],
                          ag_recv.at[half, t - 1])
            for rdma in prev_rs[half]:
                rdma.wait_send()
            for rdma in prev_ag[half]:
                rdma.wait_send()

    return pl.pallas_call(
        body,
        out_shape=jax.ShapeDtypeStruct((b, d), jnp.float32),
        in_specs=[pl.BlockSpec(memory_space=pltpu.VMEM)] * 7,
        out_specs=pl.BlockSpec(memory_space=pltpu.VMEM),
        scratch_shapes=[
            pltpu.VMEM((N_HALF, N_DEV - 1, sub, d), jnp.float32),
            pltpu.SemaphoreType.DMA((N_HALF, N_DEV - 1)),
            pltpu.SemaphoreType.DMA((N_HALF, N_DEV - 1)),
            pltpu.SemaphoreType.DMA((N_HALF, N_DEV - 1)),
            pltpu.SemaphoreType.DMA((N_HALF, N_DEV - 1)),
        ],
        compiler_params=pltpu.CompilerParams(collective_id=0),
    )(x, Win0, Wout0, Win1, Wout1, Win2, Wout2)
